# baseline (speedup 1.0000x reference)
import functools

import jax
import jax.numpy as jnp
from jax import lax
from jax.experimental import pallas as pl
from jax.experimental.pallas import tpu as pltpu
from jax.experimental.pallas import tpu_sc as plsc

V = 1000000
D = 64
B = 4096
L = 200
N = B * L

NC = 2
NS = 16
NW = NC * NS
PERW = N // NW
C = 256
BLK = 1024
NBLK = PERW // BLK
SUBS = BLK // C


def _body(seq_hbm, pos_hbm, neg_hbm, table_hbm, outp_hbm, outn_hbm,
          seq_v, posr_v, negr_v, pidx_v, nidx_v, outp_v, outn_v, sem):
    c = lax.axis_index("c")
    s = lax.axis_index("s")
    wid = s * NC + c
    lanes = lax.iota(jnp.int32, 16)

    def blk_body(blk, carry):
        bbase = wid * PERW + blk * BLK
        pltpu.sync_copy(pos_hbm.at[wid * NBLK + blk], pidx_v)
        pltpu.sync_copy(neg_hbm.at[wid * NBLK + blk], nidx_v)
        for sub in range(SUBS):
            base = bbase + sub * C
            cps = [pltpu.async_copy(seq_hbm.at[pl.ds(base, C)], seq_v, sem)]
            for j in range(C // 128):
                r = sub * (C // 128) + j
                cps.append(pltpu.async_copy(
                    table_hbm.at[pidx_v.at[r]], posr_v.at[pl.ds(j * 128, 128)], sem))
                cps.append(pltpu.async_copy(
                    table_hbm.at[nidx_v.at[r]], negr_v.at[pl.ds(j * 128, 128)], sem))
            for cp in cps:
                cp.wait()

            def group_body(grp, carry2):
                row0 = grp * 16
                resp = jnp.zeros((16,), jnp.float32)
                resn = jnp.zeros((16,), jnp.float32)
                for i in range(16):
                    p = row0 + i
                    accp = None
                    accn = None
                    for k in range(D // 16):
                        sv = seq_v[p, pl.ds(k * 16, 16)]
                        pv = posr_v[p, pl.ds(k * 16, 16)]
                        nv = negr_v[p, pl.ds(k * 16, 16)]
                        accp = sv * pv if accp is None else accp + sv * pv
                        accn = sv * nv if accn is None else accn + sv * nv
                    lane_i = lanes == i
                    resp = jnp.where(lane_i, jnp.sum(accp), resp)
                    resn = jnp.where(lane_i, jnp.sum(accn), resn)
                outp_v[pl.ds(row0, 16)] = resp
                outn_v[pl.ds(row0, 16)] = resn
                return carry2

            lax.fori_loop(0, C // 16, group_body, 0)
            pltpu.sync_copy(outp_v, outp_hbm.at[pl.ds(base, C)])
            pltpu.sync_copy(outn_v, outn_hbm.at[pl.ds(base, C)])
        return carry

    lax.fori_loop(0, NBLK, blk_body, 0)


@jax.jit
def kernel(seq, pos, neg, item_emb_table):
    seq2 = seq.reshape(N, D)
    pos2 = pos.reshape(N // BLK, 8, 128)
    neg2 = neg.reshape(N // BLK, 8, 128)
    mesh = plsc.VectorSubcoreMesh(core_axis_name="c", subcore_axis_name="s")
    run = functools.partial(
        pl.kernel,
        mesh=mesh,
        compiler_params=pltpu.CompilerParams(needs_layout_passes=False,
                                             use_tc_tiling_on_sc=False),
        out_type=[jax.ShapeDtypeStruct((N,), jnp.float32),
                  jax.ShapeDtypeStruct((N,), jnp.float32)],
        scratch_types=[
            pltpu.VMEM((C, D), jnp.float32),
            pltpu.VMEM((C, D), jnp.float32),
            pltpu.VMEM((C, D), jnp.float32),
            pltpu.VMEM((8, 128), jnp.int32),
            pltpu.VMEM((8, 128), jnp.int32),
            pltpu.VMEM((C,), jnp.float32),
            pltpu.VMEM((C,), jnp.float32),
            pltpu.SemaphoreType.DMA,
        ],
    )(_body)
    outp, outn = run(seq2, pos2, neg2, item_emb_table)
    return outp.reshape(B, L), outn.reshape(B, L)
